# baseline (speedup 1.0000x reference)
import functools

import jax
import jax.numpy as jnp
from jax import lax
from jax.experimental import pallas as pl
from jax.experimental.pallas import tpu as pltpu
from jax.experimental.pallas import tpu_sc as plsc

N = 10000
E = 320000
D = 128
D2 = 64
NG = 64
NC = 2
NS = 16
NW = NC * NS
CHUNK = 256
CPT = 40
WCH = 128
DCHUNK = 128
DCPT = 80
NB = 2
EPAD = NW * CPT * CHUNK
NP = 10240
STRIPE = NP // NS
RB = 1000
GRID = N // RB


def _mesh():
    return plsc.VectorSubcoreMesh(core_axis_name="c", subcore_axis_name="s")


@functools.partial(
    pl.kernel,
    mesh=_mesh(),
    out_type=jax.ShapeDtypeStruct((NC, NP), jnp.float32),
    scratch_types=[
        pltpu.VMEM((DCPT, DCHUNK), jnp.int32),
        pltpu.VMEM((DCHUNK,), jnp.float32),
        pltpu.VMEM((STRIPE,), jnp.float32),
        pltpu.VMEM_SHARED((NP,), jnp.float32),
    ],
)
def _sc_deg(dst_hbm, ones_hbm, zeros_hbm, out_hbm, dst_v, ones_v, zbuf,
            deg_sh):
    cid = lax.axis_index("c")
    sid = lax.axis_index("s")
    wid = cid * NS + sid

    pltpu.sync_copy(ones_hbm, ones_v)
    pltpu.sync_copy(zeros_hbm, zbuf)
    pltpu.sync_copy(zbuf, deg_sh.at[pl.ds(sid * STRIPE, STRIPE)])
    plsc.subcore_barrier()
    pltpu.sync_copy(dst_hbm.at[wid], dst_v)

    def step(g, carry):
        pltpu.sync_copy(ones_v, deg_sh.at[dst_v.at[g]], add=True)
        return carry

    lax.fori_loop(0, DCPT, step, 0)
    plsc.subcore_barrier()
    pltpu.sync_copy(deg_sh.at[pl.ds(sid * STRIPE, STRIPE)], zbuf)
    pltpu.sync_copy(zbuf, out_hbm.at[cid, pl.ds(sid * STRIPE, STRIPE)])


@functools.partial(
    pl.kernel,
    mesh=_mesh(),
    out_type=[
        jax.ShapeDtypeStruct((NC, NP, D2), jnp.float32),
        jax.ShapeDtypeStruct((NC, NP, D2), jnp.float32),
    ],
    scratch_types=[
        pltpu.VMEM((CPT, CHUNK), jnp.int32),
        pltpu.VMEM((CPT, CHUNK), jnp.int32),
        [pltpu.VMEM((CHUNK, D2), jnp.float32) for _ in range(NB)],
        [pltpu.SemaphoreType.DMA for _ in range(NB)],
        [pltpu.SemaphoreType.DMA for _ in range(NB)],
        pltpu.VMEM_SHARED((NP, D2), jnp.float32),
    ],
    compiler_params=pltpu.CompilerParams(use_tc_tiling_on_sc=False),
)
def _sc_acc(yh0_hbm, yh1_hbm, src_hbm, dst_hbm, zrow_hbm, out0_hbm, out1_hbm,
            src_v, dst_v, rows, gsem, ssem, acc_sh):
    cid = lax.axis_index("c")
    sid = lax.axis_index("s")
    wid = cid * NS + sid

    pltpu.sync_copy(src_hbm.at[wid], src_v)
    pltpu.sync_copy(dst_hbm.at[wid], dst_v)

    def wait_gather(g, b):
        pltpu.make_async_copy(
            yh0_hbm.at[src_v.at[g]], rows[b], gsem[b]).wait()

    def wait_scatter(g, b):
        pltpu.make_async_copy(
            rows[b], acc_sh.at[dst_v.at[g]], ssem[b]).wait()

    for y_hbm, out_hbm in ((yh0_hbm, out0_hbm), (yh1_hbm, out1_hbm)):
        wbuf = rows[0].at[pl.ds(0, WCH)]
        pltpu.sync_copy(zrow_hbm, wbuf)
        for k in range(STRIPE // WCH):
            pltpu.sync_copy(
                wbuf, acc_sh.at[pl.ds(sid * STRIPE + k * WCH, WCH)])
        plsc.subcore_barrier()

        for b in range(NB):
            pltpu.async_copy(y_hbm.at[src_v.at[b]], rows[b], gsem[b])

        def step(i, carry):
            g = NB * i
            for b in range(NB):
                wait_gather(g + b, b)
                pltpu.async_copy(
                    rows[b], acc_sh.at[dst_v.at[g + b]], ssem[b], add=True)
            for b in range(NB):
                wait_scatter(g + b, b)
                pltpu.async_copy(
                    y_hbm.at[src_v.at[g + NB + b]], rows[b], gsem[b])
            return carry

        lax.fori_loop(0, CPT // NB - 1, step, 0)
        g = CPT - NB
        for b in range(NB):
            wait_gather(g + b, b)
            pltpu.async_copy(
                rows[b], acc_sh.at[dst_v.at[g + b]], ssem[b], add=True)
        for b in range(NB):
            wait_scatter(g + b, b)
        plsc.subcore_barrier()

        for k in range(STRIPE // WCH):
            off = sid * STRIPE + k * WCH
            pltpu.sync_copy(acc_sh.at[pl.ds(off, WCH)], wbuf)
            pltpu.sync_copy(wbuf, out_hbm.at[cid, pl.ds(off, WCH)])
        plsc.subcore_barrier()


def _dinv_block(dega, degb):
    return lax.rsqrt(dega + degb + 1.0)


def _pre_body(x_ref, w_ref, dega_ref, degb_ref, yh0_ref, yh1_ref):
    dinv = _dinv_block(dega_ref[...], degb_ref[...])
    y = jnp.dot(x_ref[...], w_ref[...],
                preferred_element_type=jnp.float32) * dinv
    yh0_ref[...] = y[:, :D2]
    yh1_ref[...] = y[:, D2:]


def _tc_pre(x, W, dega, degb):
    return pl.pallas_call(
        _pre_body,
        grid=(GRID,),
        in_specs=[
            pl.BlockSpec((RB, D), lambda i: (i, 0)),
            pl.BlockSpec((D, D), lambda i: (0, 0)),
            pl.BlockSpec((RB, 1), lambda i: (i, 0)),
            pl.BlockSpec((RB, 1), lambda i: (i, 0)),
        ],
        out_specs=[
            pl.BlockSpec((RB, D2), lambda i: (i, 0)),
            pl.BlockSpec((RB, D2), lambda i: (i, 0)),
        ],
        out_shape=[
            jax.ShapeDtypeStruct((N, D2), jnp.float32),
            jax.ShapeDtypeStruct((N, D2), jnp.float32),
        ],
    )(x, W, dega, degb)


def _pool_accum(g_ref, batch_blk, z_blk):
    onehot = (batch_blk == lax.broadcasted_iota(
        jnp.int32, (RB, NG), 1)).astype(jnp.float32)
    part = lax.dot_general(onehot, z_blk, (((0,), (0,)), ((), ())),
                           preferred_element_type=jnp.float32)

    @pl.when(pl.program_id(0) == 0)
    def _():
        g_ref[...] = jnp.zeros_like(g_ref)

    g_ref[...] += part


def _gcn_combine(a00_ref, a01_ref, a10_ref, a11_ref, yh0_ref, yh1_ref,
                 dinv, b_ref, alpha_ref):
    lo = a00_ref[...] + a10_ref[...] + yh0_ref[...]
    hi = a01_ref[...] + a11_ref[...] + yh1_ref[...]
    z = dinv * jnp.concatenate([lo, hi], axis=1) + b_ref[...]
    return jnp.where(z >= 0.0, z, alpha_ref[...] * z)


def _mid_body(a00_ref, a01_ref, a10_ref, a11_ref, yh0_ref, yh1_ref,
              dega_ref, degb_ref, b_ref, alpha_ref, w2_ref, batch_ref,
              y2h0_ref, y2h1_ref, g_ref):
    dinv = _dinv_block(dega_ref[...], degb_ref[...])
    z = _gcn_combine(a00_ref, a01_ref, a10_ref, a11_ref, yh0_ref, yh1_ref,
                     dinv, b_ref, alpha_ref)
    y2 = jnp.dot(z, w2_ref[...], preferred_element_type=jnp.float32) * dinv
    y2h0_ref[...] = y2[:, :D2]
    y2h1_ref[...] = y2[:, D2:]
    _pool_accum(g_ref, batch_ref[...], z)


def _tc_mid(a00, a01, a10, a11, yh0, yh1, dega, degb, b1, alpha, W2,
            batch2d):
    return pl.pallas_call(
        _mid_body,
        grid=(GRID,),
        in_specs=[
            pl.BlockSpec((RB, D2), lambda i: (i, 0)),
            pl.BlockSpec((RB, D2), lambda i: (i, 0)),
            pl.BlockSpec((RB, D2), lambda i: (i, 0)),
            pl.BlockSpec((RB, D2), lambda i: (i, 0)),
            pl.BlockSpec((RB, D2), lambda i: (i, 0)),
            pl.BlockSpec((RB, D2), lambda i: (i, 0)),
            pl.BlockSpec((RB, 1), lambda i: (i, 0)),
            pl.BlockSpec((RB, 1), lambda i: (i, 0)),
            pl.BlockSpec((D,), lambda i: (0,)),
            pl.BlockSpec((D,), lambda i: (0,)),
            pl.BlockSpec((D, D), lambda i: (0, 0)),
            pl.BlockSpec((RB, 1), lambda i: (i, 0)),
        ],
        out_specs=[
            pl.BlockSpec((RB, D2), lambda i: (i, 0)),
            pl.BlockSpec((RB, D2), lambda i: (i, 0)),
            pl.BlockSpec((NG, D), lambda i: (0, 0)),
        ],
        out_shape=[
            jax.ShapeDtypeStruct((N, D2), jnp.float32),
            jax.ShapeDtypeStruct((N, D2), jnp.float32),
            jax.ShapeDtypeStruct((NG, D), jnp.float32),
        ],
    )(a00, a01, a10, a11, yh0, yh1, dega, degb, b1, alpha, W2, batch2d)


def _post_body(a00_ref, a01_ref, a10_ref, a11_ref, yh0_ref, yh1_ref,
               dega_ref, degb_ref, b_ref, alpha_ref, batch_ref,
               z_ref, g_ref):
    dinv = _dinv_block(dega_ref[...], degb_ref[...])
    z = _gcn_combine(a00_ref, a01_ref, a10_ref, a11_ref, yh0_ref, yh1_ref,
                     dinv, b_ref, alpha_ref)
    z_ref[...] = z
    _pool_accum(g_ref, batch_ref[...], z)


def _tc_post(a00, a01, a10, a11, yh0, yh1, dega, degb, b2, alpha, batch2d):
    return pl.pallas_call(
        _post_body,
        grid=(GRID,),
        in_specs=[
            pl.BlockSpec((RB, D2), lambda i: (i, 0)),
            pl.BlockSpec((RB, D2), lambda i: (i, 0)),
            pl.BlockSpec((RB, D2), lambda i: (i, 0)),
            pl.BlockSpec((RB, D2), lambda i: (i, 0)),
            pl.BlockSpec((RB, D2), lambda i: (i, 0)),
            pl.BlockSpec((RB, D2), lambda i: (i, 0)),
            pl.BlockSpec((RB, 1), lambda i: (i, 0)),
            pl.BlockSpec((RB, 1), lambda i: (i, 0)),
            pl.BlockSpec((D,), lambda i: (0,)),
            pl.BlockSpec((D,), lambda i: (0,)),
            pl.BlockSpec((RB, 1), lambda i: (i, 0)),
        ],
        out_specs=[
            pl.BlockSpec((RB, D), lambda i: (i, 0)),
            pl.BlockSpec((NG, D), lambda i: (0, 0)),
        ],
        out_shape=[
            jax.ShapeDtypeStruct((N, D), jnp.float32),
            jax.ShapeDtypeStruct((NG, D), jnp.float32),
        ],
    )(a00, a01, a10, a11, yh0, yh1, dega, degb, b2, alpha, batch2d)


def kernel(x, edge_index, batch, W1, b1, W2, b2, alpha):
    pad = EPAD - E
    pad_src = jnp.arange(pad, dtype=jnp.int32) % N
    pad_dst = N + (jnp.arange(pad, dtype=jnp.int32) % (NP - N))
    srcp = jnp.concatenate([edge_index[0], pad_src]).reshape(NW, CPT, CHUNK)
    dstp = jnp.concatenate([edge_index[1], pad_dst]).reshape(NW, CPT, CHUNK)

    ones_c = jnp.ones((DCHUNK,), jnp.float32)
    zeros_s = jnp.zeros((STRIPE,), jnp.float32)
    zrow = jnp.zeros((WCH, D2), jnp.float32)

    deg2 = _sc_deg(dstp.reshape(NW, DCPT, DCHUNK), ones_c, zeros_s)
    dega = deg2[0].reshape(NP, 1)
    degb = deg2[1].reshape(NP, 1)
    batch2d = batch[:, None]

    y1h0, y1h1 = _tc_pre(x, W1, dega, degb)
    acc1h0, acc1h1 = _sc_acc(y1h0, y1h1, srcp, dstp, zrow)
    y2h0, y2h1, g1 = _tc_mid(acc1h0[0], acc1h1[0], acc1h0[1], acc1h1[1],
                             y1h0, y1h1, dega, degb, b1, alpha, W2, batch2d)
    acc2h0, acc2h1 = _sc_acc(y2h0, y2h1, srcp, dstp, zrow)
    z2, g2 = _tc_post(acc2h0[0], acc2h1[0], acc2h0[1], acc2h1[1],
                      y2h0, y2h1, dega, degb, b2, alpha, batch2d)
    return z2, jnp.concatenate([g1, g2], axis=1)


# speedup vs baseline: 1.1599x; 1.1599x over previous
import functools

import jax
import jax.numpy as jnp
from jax import lax
from jax.experimental import pallas as pl
from jax.experimental.pallas import tpu as pltpu
from jax.experimental.pallas import tpu_sc as plsc

N = 10000
E = 320000
D = 128
D2 = 64
NG = 64
NC = 2
NS = 16
NW = NC * NS
CHUNK = 128
CPT = 80
NB = 5
EPAD = NW * CPT * CHUNK
NP = 10240
STRIPE = NP // NS
RB = 1000
GRID = N // RB


def _mesh():
    return plsc.VectorSubcoreMesh(core_axis_name="c", subcore_axis_name="s")


@functools.partial(
    pl.kernel,
    mesh=_mesh(),
    out_type=jax.ShapeDtypeStruct((NC, NP), jnp.float32),
    scratch_types=[
        pltpu.VMEM((CPT, CHUNK), jnp.int32),
        pltpu.VMEM((CHUNK,), jnp.float32),
        pltpu.VMEM((STRIPE,), jnp.float32),
        pltpu.VMEM_SHARED((NP,), jnp.float32),
    ],
)
def _sc_deg(dst_hbm, ones_hbm, zeros_hbm, out_hbm, dst_v, ones_v, zbuf,
            deg_sh):
    cid = lax.axis_index("c")
    sid = lax.axis_index("s")
    wid = cid * NS + sid

    pltpu.sync_copy(ones_hbm, ones_v)
    pltpu.sync_copy(zeros_hbm, zbuf)
    pltpu.sync_copy(zbuf, deg_sh.at[pl.ds(sid * STRIPE, STRIPE)])
    plsc.subcore_barrier()
    pltpu.sync_copy(dst_hbm.at[wid], dst_v)

    def step(g, carry):
        pltpu.sync_copy(ones_v, deg_sh.at[dst_v.at[g]], add=True)
        return carry

    lax.fori_loop(0, CPT, step, 0)
    plsc.subcore_barrier()
    pltpu.sync_copy(deg_sh.at[pl.ds(sid * STRIPE, STRIPE)], zbuf)
    pltpu.sync_copy(zbuf, out_hbm.at[cid, pl.ds(sid * STRIPE, STRIPE)])


@functools.partial(
    pl.kernel,
    mesh=_mesh(),
    out_type=[
        jax.ShapeDtypeStruct((NC, NP, D2), jnp.float32),
        jax.ShapeDtypeStruct((NC, NP, D2), jnp.float32),
    ],
    scratch_types=[
        pltpu.VMEM((CPT, CHUNK), jnp.int32),
        pltpu.VMEM((CPT, CHUNK), jnp.int32),
        [pltpu.VMEM((CHUNK, D2), jnp.float32) for _ in range(NB)],
        [pltpu.SemaphoreType.DMA for _ in range(NB)],
        [pltpu.SemaphoreType.DMA for _ in range(NB)],
        pltpu.VMEM_SHARED((NP, D2), jnp.float32),
    ],
    compiler_params=pltpu.CompilerParams(use_tc_tiling_on_sc=False),
)
def _sc_acc(yh0_hbm, yh1_hbm, src_hbm, dst_hbm, zrow_hbm, out0_hbm, out1_hbm,
            src_v, dst_v, rows, gsem, ssem, acc_sh):
    cid = lax.axis_index("c")
    sid = lax.axis_index("s")
    wid = cid * NS + sid

    pltpu.sync_copy(src_hbm.at[wid], src_v)
    pltpu.sync_copy(dst_hbm.at[wid], dst_v)

    def wait_gather(g, b):
        pltpu.make_async_copy(
            yh0_hbm.at[src_v.at[g]], rows[b], gsem[b]).wait()

    def wait_scatter(g, b):
        pltpu.make_async_copy(
            rows[b], acc_sh.at[dst_v.at[g]], ssem[b]).wait()

    for y_hbm, out_hbm in ((yh0_hbm, out0_hbm), (yh1_hbm, out1_hbm)):
        pltpu.sync_copy(zrow_hbm, rows[0])
        for k in range(STRIPE // CHUNK):
            pltpu.sync_copy(
                rows[0], acc_sh.at[pl.ds(sid * STRIPE + k * CHUNK, CHUNK)])
        plsc.subcore_barrier()

        for b in range(NB):
            pltpu.async_copy(y_hbm.at[src_v.at[b]], rows[b], gsem[b])

        def step(i, carry):
            g = NB * i
            for b in range(NB):
                wait_gather(g + b, b)
                pltpu.async_copy(
                    rows[b], acc_sh.at[dst_v.at[g + b]], ssem[b], add=True)
            for b in range(NB):
                wait_scatter(g + b, b)
                pltpu.async_copy(
                    y_hbm.at[src_v.at[g + NB + b]], rows[b], gsem[b])
            return carry

        lax.fori_loop(0, CPT // NB - 1, step, 0)
        g = CPT - NB
        for b in range(NB):
            wait_gather(g + b, b)
            pltpu.async_copy(
                rows[b], acc_sh.at[dst_v.at[g + b]], ssem[b], add=True)
        for b in range(NB):
            wait_scatter(g + b, b)
        plsc.subcore_barrier()

        for k in range(STRIPE // CHUNK):
            off = sid * STRIPE + k * CHUNK
            pltpu.sync_copy(acc_sh.at[pl.ds(off, CHUNK)], rows[0])
            pltpu.sync_copy(rows[0], out_hbm.at[cid, pl.ds(off, CHUNK)])


def _dinv_block(dega, degb):
    return lax.rsqrt(dega + degb + 1.0)


def _pre_body(x_ref, w_ref, dega_ref, degb_ref, yh0_ref, yh1_ref):
    dinv = _dinv_block(dega_ref[...], degb_ref[...])
    y = jnp.dot(x_ref[...], w_ref[...],
                preferred_element_type=jnp.float32) * dinv
    yh0_ref[...] = y[:, :D2]
    yh1_ref[...] = y[:, D2:]


def _tc_pre(x, W, dega, degb):
    return pl.pallas_call(
        _pre_body,
        grid=(GRID,),
        in_specs=[
            pl.BlockSpec((RB, D), lambda i: (i, 0)),
            pl.BlockSpec((D, D), lambda i: (0, 0)),
            pl.BlockSpec((RB, 1), lambda i: (i, 0)),
            pl.BlockSpec((RB, 1), lambda i: (i, 0)),
        ],
        out_specs=[
            pl.BlockSpec((RB, D2), lambda i: (i, 0)),
            pl.BlockSpec((RB, D2), lambda i: (i, 0)),
        ],
        out_shape=[
            jax.ShapeDtypeStruct((N, D2), jnp.float32),
            jax.ShapeDtypeStruct((N, D2), jnp.float32),
        ],
    )(x, W, dega, degb)


def _pool_accum(g_ref, batch_blk, z_blk):
    onehot = (batch_blk == lax.broadcasted_iota(
        jnp.int32, (RB, NG), 1)).astype(jnp.float32)
    part = lax.dot_general(onehot, z_blk, (((0,), (0,)), ((), ())),
                           preferred_element_type=jnp.float32)

    @pl.when(pl.program_id(0) == 0)
    def _():
        g_ref[...] = jnp.zeros_like(g_ref)

    g_ref[...] += part


def _gcn_combine(a00_ref, a01_ref, a10_ref, a11_ref, yh0_ref, yh1_ref,
                 dinv, b_ref, alpha_ref):
    lo = a00_ref[...] + a10_ref[...] + yh0_ref[...]
    hi = a01_ref[...] + a11_ref[...] + yh1_ref[...]
    z = dinv * jnp.concatenate([lo, hi], axis=1) + b_ref[...]
    return jnp.where(z >= 0.0, z, alpha_ref[...] * z)


def _mid_body(a00_ref, a01_ref, a10_ref, a11_ref, yh0_ref, yh1_ref,
              dega_ref, degb_ref, b_ref, alpha_ref, w2_ref, batch_ref,
              y2h0_ref, y2h1_ref, g_ref):
    dinv = _dinv_block(dega_ref[...], degb_ref[...])
    z = _gcn_combine(a00_ref, a01_ref, a10_ref, a11_ref, yh0_ref, yh1_ref,
                     dinv, b_ref, alpha_ref)
    y2 = jnp.dot(z, w2_ref[...], preferred_element_type=jnp.float32) * dinv
    y2h0_ref[...] = y2[:, :D2]
    y2h1_ref[...] = y2[:, D2:]
    _pool_accum(g_ref, batch_ref[...], z)


def _tc_mid(a00, a01, a10, a11, yh0, yh1, dega, degb, b1, alpha, W2,
            batch2d):
    return pl.pallas_call(
        _mid_body,
        grid=(GRID,),
        in_specs=[
            pl.BlockSpec((RB, D2), lambda i: (i, 0)),
            pl.BlockSpec((RB, D2), lambda i: (i, 0)),
            pl.BlockSpec((RB, D2), lambda i: (i, 0)),
            pl.BlockSpec((RB, D2), lambda i: (i, 0)),
            pl.BlockSpec((RB, D2), lambda i: (i, 0)),
            pl.BlockSpec((RB, D2), lambda i: (i, 0)),
            pl.BlockSpec((RB, 1), lambda i: (i, 0)),
            pl.BlockSpec((RB, 1), lambda i: (i, 0)),
            pl.BlockSpec((D,), lambda i: (0,)),
            pl.BlockSpec((D,), lambda i: (0,)),
            pl.BlockSpec((D, D), lambda i: (0, 0)),
            pl.BlockSpec((RB, 1), lambda i: (i, 0)),
        ],
        out_specs=[
            pl.BlockSpec((RB, D2), lambda i: (i, 0)),
            pl.BlockSpec((RB, D2), lambda i: (i, 0)),
            pl.BlockSpec((NG, D), lambda i: (0, 0)),
        ],
        out_shape=[
            jax.ShapeDtypeStruct((N, D2), jnp.float32),
            jax.ShapeDtypeStruct((N, D2), jnp.float32),
            jax.ShapeDtypeStruct((NG, D), jnp.float32),
        ],
    )(a00, a01, a10, a11, yh0, yh1, dega, degb, b1, alpha, W2, batch2d)


def _post_body(a00_ref, a01_ref, a10_ref, a11_ref, yh0_ref, yh1_ref,
               dega_ref, degb_ref, b_ref, alpha_ref, batch_ref,
               z_ref, g_ref):
    dinv = _dinv_block(dega_ref[...], degb_ref[...])
    z = _gcn_combine(a00_ref, a01_ref, a10_ref, a11_ref, yh0_ref, yh1_ref,
                     dinv, b_ref, alpha_ref)
    z_ref[...] = z
    _pool_accum(g_ref, batch_ref[...], z)


def _tc_post(a00, a01, a10, a11, yh0, yh1, dega, degb, b2, alpha, batch2d):
    return pl.pallas_call(
        _post_body,
        grid=(GRID,),
        in_specs=[
            pl.BlockSpec((RB, D2), lambda i: (i, 0)),
            pl.BlockSpec((RB, D2), lambda i: (i, 0)),
            pl.BlockSpec((RB, D2), lambda i: (i, 0)),
            pl.BlockSpec((RB, D2), lambda i: (i, 0)),
            pl.BlockSpec((RB, D2), lambda i: (i, 0)),
            pl.BlockSpec((RB, D2), lambda i: (i, 0)),
            pl.BlockSpec((RB, 1), lambda i: (i, 0)),
            pl.BlockSpec((RB, 1), lambda i: (i, 0)),
            pl.BlockSpec((D,), lambda i: (0,)),
            pl.BlockSpec((D,), lambda i: (0,)),
            pl.BlockSpec((RB, 1), lambda i: (i, 0)),
        ],
        out_specs=[
            pl.BlockSpec((RB, D), lambda i: (i, 0)),
            pl.BlockSpec((NG, D), lambda i: (0, 0)),
        ],
        out_shape=[
            jax.ShapeDtypeStruct((N, D), jnp.float32),
            jax.ShapeDtypeStruct((NG, D), jnp.float32),
        ],
    )(a00, a01, a10, a11, yh0, yh1, dega, degb, b2, alpha, batch2d)


def kernel(x, edge_index, batch, W1, b1, W2, b2, alpha):
    pad = EPAD - E
    pad_src = jnp.arange(pad, dtype=jnp.int32) % N
    pad_dst = N + (jnp.arange(pad, dtype=jnp.int32) % (NP - N))
    srcp = jnp.concatenate([edge_index[0], pad_src]).reshape(NW, CPT, CHUNK)
    dstp = jnp.concatenate([edge_index[1], pad_dst]).reshape(NW, CPT, CHUNK)

    ones_c = jnp.ones((CHUNK,), jnp.float32)
    zeros_s = jnp.zeros((STRIPE,), jnp.float32)
    zrow = jnp.zeros((CHUNK, D2), jnp.float32)

    deg2 = _sc_deg(dstp, ones_c, zeros_s)
    dega = deg2[0].reshape(NP, 1)
    degb = deg2[1].reshape(NP, 1)
    batch2d = batch[:, None]

    y1h0, y1h1 = _tc_pre(x, W1, dega, degb)
    acc1h0, acc1h1 = _sc_acc(y1h0, y1h1, srcp, dstp, zrow)
    y2h0, y2h1, g1 = _tc_mid(acc1h0[0], acc1h1[0], acc1h0[1], acc1h1[1],
                             y1h0, y1h1, dega, degb, b1, alpha, W2, batch2d)
    acc2h0, acc2h1 = _sc_acc(y2h0, y2h1, srcp, dstp, zrow)
    z2, g2 = _tc_post(acc2h0[0], acc2h1[0], acc2h0[1], acc2h1[1],
                      y2h0, y2h1, dega, degb, b2, alpha, batch2d)
    return z2, jnp.concatenate([g1, g2], axis=1)
